# baseline (speedup 1.0000x reference)
import functools

import jax
import jax.numpy as jnp
from jax import lax
from jax.experimental import pallas as pl
from jax.experimental.pallas import tpu as pltpu
from jax.experimental.pallas import tpu_sc as plsc

DIM = 128
NUM_CTX = 5
LANES = 16
VREGS = DIM // LANES

NC = 2
NS = 16
NW = NC * NS


def _sc_dot_kernel(batch):
    b_per_w = batch // NW
    cb = 64
    nch = b_per_w // cb
    crows = cb * NUM_CTX

    mesh = plsc.VectorSubcoreMesh(core_axis_name="c", subcore_axis_name="s")

    @functools.partial(
        pl.kernel,
        mesh=mesh,
        out_type=jax.ShapeDtypeStruct((NUM_CTX, batch), jnp.float32),
        compiler_params=pltpu.CompilerParams(needs_layout_passes=False),
        scratch_types=[
            pltpu.VMEM((b_per_w,), jnp.int32),
            pltpu.VMEM((b_per_w * NUM_CTX,), jnp.int32),
            pltpu.VMEM((2, cb, DIM), jnp.float32),
            pltpu.VMEM((2, crows, DIM), jnp.float32),
            pltpu.VMEM((NUM_CTX, cb), jnp.float32),
            pltpu.VMEM((NUM_CTX, cb), jnp.float32),
            pltpu.VMEM((crows * LANES,), jnp.float32),
            pltpu.SemaphoreType.DMA,
            pltpu.SemaphoreType.DMA,
            pltpu.SemaphoreType.DMA,
        ],
    )
    def kern(t_idx, c_idx, w_t, w_c, out, tix, cix, tbuf, cbuf, obuf0, obuf1,
             tr, sem0, sem1, osem):
        obufs = (obuf0, obuf1)
        wid = lax.axis_index("s") * NC + lax.axis_index("c")
        base = wid * b_per_w
        sems = (sem0, sem1)

        idx_ds = [pltpu.async_copy(t_idx.at[pl.ds(base, b_per_w)], tix, osem)]
        for n in range(NUM_CTX):
            idx_ds.append(pltpu.async_copy(
                c_idx.at[pl.ds(n * batch + base, b_per_w)],
                cix.at[pl.ds(n * b_per_w, b_per_w)],
                osem))
        for d in idx_ds:
            d.wait()

        def fire(ch, slot):
            sem = sems[slot]
            ds = [pltpu.async_copy(
                w_t.at[tix.at[pl.ds(ch * cb, cb)]], tbuf.at[slot], sem)]
            for n in range(NUM_CTX):
                src = cix.at[pl.ds(n * b_per_w + ch * cb, cb)]
                dst = cbuf.at[slot, pl.ds(n * cb, cb)]
                ds.append(pltpu.async_copy(w_c.at[src], dst, sem))
            return ds

        iota = lax.iota(jnp.int32, LANES)

        def compute(ch, slot):
            @plsc.parallel_loop(0, cb, unroll=2)
            def body(b):
                q = b >> 4
                l = b & (LANES - 1)
                tb = q * (LANES * LANES * NUM_CTX) + l * LANES
                we = [tbuf[slot, b, pl.ds(k * LANES, LANES)]
                      for k in range(VREGS)]
                for n in range(NUM_CTX):
                    row = n * cb + b
                    p = [cbuf[slot, row, pl.ds(k * LANES, LANES)] * we[k]
                         for k in range(VREGS)]
                    acc = (((p[0] + p[1]) + (p[2] + p[3]))
                           + ((p[4] + p[5]) + (p[6] + p[7])))
                    tr[pl.ds(tb + n * (LANES * LANES), LANES)] = acc

            @plsc.parallel_loop(0, cb // LANES)
            def qbody(q):
                for n in range(NUM_CTX):
                    tbase = (q * (LANES * LANES * NUM_CTX)
                             + n * (LANES * LANES) + iota * LANES)
                    c = [plsc.load_gather(tr, [tbase + j])
                         for j in range(LANES)]
                    for step in (8, 4, 2, 1):
                        c = [c[j] + c[j + step] for j in range(step)]
                    obufs[slot][n, pl.ds(q * LANES, LANES)] = c[0]

        pend = fire(0, 0)
        out_ds = [None] * nch
        for ch in range(nch):
            slot = ch % 2
            nxt = fire(ch + 1, 1 - slot) if ch + 1 < nch else None
            for d in pend:
                d.wait()
            if ch >= 2:
                for d in out_ds[ch - 2]:
                    d.wait()
            compute(ch, slot)
            out_ds[ch] = [
                pltpu.async_copy(
                    obufs[slot].at[n],
                    out.at[n, pl.ds(base + ch * cb, cb)],
                    osem)
                for n in range(NUM_CTX)]
            pend = nxt
        for ch in range(max(0, nch - 2), nch):
            for d in out_ds[ch]:
                d.wait()

    return kern


def kernel(target, context, W_target, W_context):
    batch = target.shape[0]

    t_idx = target.reshape(batch)
    c_idx = context.reshape(batch, NUM_CTX).T.reshape(NUM_CTX * batch)

    out = _sc_dot_kernel(batch)(t_idx, c_idx, W_target, W_context)
    return out.T


# speedup vs baseline: 1.0228x; 1.0228x over previous
import functools

import jax
import jax.numpy as jnp
from jax import lax
from jax.experimental import pallas as pl
from jax.experimental.pallas import tpu as pltpu
from jax.experimental.pallas import tpu_sc as plsc

DIM = 128
NUM_CTX = 5
LANES = 16
VREGS = DIM // LANES

NC = 2
NS = 16
NW = NC * NS


def _sc_dot_kernel(batch):
    b_per_w = batch // NW
    cb = 64
    nch = b_per_w // cb
    crows = cb * NUM_CTX

    mesh = plsc.VectorSubcoreMesh(core_axis_name="c", subcore_axis_name="s")

    @functools.partial(
        pl.kernel,
        mesh=mesh,
        out_type=jax.ShapeDtypeStruct((NUM_CTX, batch), jnp.float32),
        compiler_params=pltpu.CompilerParams(needs_layout_passes=False),
        scratch_types=[
            pltpu.VMEM((b_per_w,), jnp.int32),
            pltpu.VMEM((b_per_w * NUM_CTX,), jnp.int32),
            pltpu.VMEM((2, cb, DIM), jnp.float32),
            pltpu.VMEM((2, crows, DIM), jnp.float32),
            pltpu.VMEM((NUM_CTX, cb), jnp.float32),
            pltpu.VMEM((NUM_CTX, cb), jnp.float32),
            pltpu.VMEM((crows * LANES,), jnp.float32),
            pltpu.SemaphoreType.DMA,
            pltpu.SemaphoreType.DMA,
            pltpu.SemaphoreType.DMA,
        ],
    )
    def kern(t_idx, c_idx, w_t, w_c, out, tix, cix, tbuf, cbuf, obuf0, obuf1,
             tr, sem0, sem1, osem):
        obufs = (obuf0, obuf1)
        wid = lax.axis_index("s") * NC + lax.axis_index("c")
        base = wid * b_per_w
        sems = (sem0, sem1)

        idx_ds = [pltpu.async_copy(t_idx.at[pl.ds(base, b_per_w)], tix, osem)]
        for n in range(NUM_CTX):
            idx_ds.append(pltpu.async_copy(
                c_idx.at[pl.ds(n * batch + base, b_per_w)],
                cix.at[pl.ds(n * b_per_w, b_per_w)],
                osem))
        for d in idx_ds:
            d.wait()

        def fire(ch, slot):
            sem = sems[slot]
            ds = [pltpu.async_copy(
                w_t.at[tix.at[pl.ds(ch * cb, cb)]], tbuf.at[slot], sem)]
            for n in range(NUM_CTX):
                src = cix.at[pl.ds(n * b_per_w + ch * cb, cb)]
                dst = cbuf.at[slot, pl.ds(n * cb, cb)]
                ds.append(pltpu.async_copy(w_c.at[src], dst, sem))
            return ds

        iota = lax.iota(jnp.int32, LANES)

        def compute(ch, slot):
            @plsc.parallel_loop(0, cb)
            def body(b):
                q = b >> 4
                l = b & (LANES - 1)
                tb = q * (LANES * LANES * NUM_CTX) + l * LANES
                we = [tbuf[slot, b, pl.ds(k * LANES, LANES)]
                      for k in range(VREGS)]
                for n in range(NUM_CTX):
                    row = n * cb + b
                    p = [cbuf[slot, row, pl.ds(k * LANES, LANES)] * we[k]
                         for k in range(VREGS)]
                    acc = (((p[0] + p[1]) + (p[2] + p[3]))
                           + ((p[4] + p[5]) + (p[6] + p[7])))
                    tr[pl.ds(tb + n * (LANES * LANES), LANES)] = acc

            @plsc.parallel_loop(0, cb // LANES)
            def qbody(q):
                for n in range(NUM_CTX):
                    tbase = (q * (LANES * LANES * NUM_CTX)
                             + n * (LANES * LANES) + iota * LANES)
                    c = [plsc.load_gather(tr, [tbase + j])
                         for j in range(LANES)]
                    for step in (8, 4, 2, 1):
                        c = [c[j] + c[j + step] for j in range(step)]
                    obufs[slot][n, pl.ds(q * LANES, LANES)] = c[0]

        pend = fire(0, 0)
        out_ds = [None] * nch
        for ch in range(nch):
            slot = ch % 2
            nxt = fire(ch + 1, 1 - slot) if ch + 1 < nch else None
            for d in pend:
                d.wait()
            if ch >= 2:
                for d in out_ds[ch - 2]:
                    d.wait()
            compute(ch, slot)
            out_ds[ch] = [
                pltpu.async_copy(
                    obufs[slot].at[n],
                    out.at[n, pl.ds(base + ch * cb, cb)],
                    osem)
                for n in range(NUM_CTX)]
            pend = nxt
        for ch in range(max(0, nch - 2), nch):
            for d in out_ds[ch]:
                d.wait()

    return kern


def kernel(target, context, W_target, W_context):
    batch = target.shape[0]

    t_idx = target.reshape(batch)
    c_idx = context.reshape(batch, NUM_CTX).T.reshape(NUM_CTX * batch)

    out = _sc_dot_kernel(batch)(t_idx, c_idx, W_target, W_context)
    return out.T
